# baseline (speedup 1.0000x reference)
import functools

import jax
import jax.numpy as jnp
from jax import lax
from jax.experimental import pallas as pl
from jax.experimental.pallas import tpu as pltpu
from jax.experimental.pallas import tpu_sc as plsc

BATCH = 16384
N_FIELDS = 26
EMBED_DIM = 16

NUM_CORES = 2
NUM_SUBCORES = 16
NW = NUM_CORES * NUM_SUBCORES

B_PER_W = BATCH // NW
BT = 128
K_BLOCKS = B_PER_W // BT
IDX_ROWS = B_PER_W * N_FIELDS // 128
BE = 16
RPE = BE * N_FIELDS
STREAM = 104
S_PER_E = RPE // STREAM
N_E = B_PER_W // BE
E_PER_BLOCK = BT // BE

_mesh = plsc.VectorSubcoreMesh(
    core_axis_name="c", subcore_axis_name="s",
    num_cores=NUM_CORES, num_subcores=NUM_SUBCORES,
)


@functools.partial(
    pl.kernel,
    out_type=jax.ShapeDtypeStruct(
        (N_FIELDS, EMBED_DIM // 8, BATCH // BT, 8, BT), jnp.float32),
    mesh=_mesh,
    compiler_params=pltpu.CompilerParams(
        use_tc_tiling_on_sc=True, needs_layout_passes=False),
    scratch_types=[
        pltpu.VMEM((IDX_ROWS, 128), jnp.int32),
        pltpu.VMEM((RPE,), jnp.int32),
        pltpu.VMEM((RPE, 128), jnp.float32),
        pltpu.VMEM((N_FIELDS, EMBED_DIM // 8, 8, BT), jnp.float32),
        pltpu.SemaphoreType.DMA,
    ],
)
def _gather_rows(idx_hbm, table8_hbm, out_hbm, idx_v, tid_v, rows_v, stage_v,
                 sem):
    wid = lax.axis_index("s") * NUM_CORES + lax.axis_index("c")
    pltpu.sync_copy(idx_hbm.at[pl.ds(wid * IDX_ROWS, IDX_ROWS)], idx_v)
    lane = jnp.arange(16, dtype=jnp.int32)

    @pl.loop(0, N_E)
    def _step(m):
        @pl.loop(0, RPE // 16)
        def _gen(c):
            flat = m * RPE + c * 16 + lane
            v = plsc.load_gather(idx_v, [flat >> 7, flat & 127])
            tid_v[pl.ds(c * 16, 16)] = v >> 3

        copies = []
        for t in range(S_PER_E):
            copies.append(
                pltpu.async_copy(
                    table8_hbm.at[tid_v.at[pl.ds(t * STREAM, STREAM)]],
                    rows_v.at[pl.ds(t * STREAM, STREAM)],
                    sem,
                )
            )
        for c in copies:
            c.wait()

        @plsc.parallel_loop(0, N_FIELDS, unroll=2)
        def _field(f):
            slot = lane * N_FIELDS + f
            flat = m * RPE + slot
            low = plsc.load_gather(idx_v, [flat >> 7, flat & 127]) & 7
            w0 = low * 16
            for e in range(EMBED_DIM):
                et, es = divmod(e, 8)
                v = plsc.load_gather(rows_v, [slot, w0 + e])
                stage_v[f, et, es, pl.ds((m % E_PER_BLOCK) * BE, 16)] = v

        @pl.when(m % E_PER_BLOCK == E_PER_BLOCK - 1)
        def _writeback():
            pltpu.sync_copy(
                stage_v,
                out_hbm.at[:, :, wid * K_BLOCKS + m // E_PER_BLOCK],
            )


def kernel(inputs, table):
    idx = inputs.reshape(BATCH * N_FIELDS // 128, 128)
    out5 = _gather_rows(idx, table.reshape(125000, 128))
    return out5.transpose(2, 4, 0, 1, 3).reshape(BATCH, N_FIELDS, EMBED_DIM)


# speedup vs baseline: 1.0668x; 1.0668x over previous
import functools

import jax
import jax.numpy as jnp
from jax import lax
from jax.experimental import pallas as pl
from jax.experimental.pallas import tpu as pltpu
from jax.experimental.pallas import tpu_sc as plsc

BATCH = 16384
N_FIELDS = 26
EMBED_DIM = 16

NUM_CORES = 2
NUM_SUBCORES = 16
NW = NUM_CORES * NUM_SUBCORES

B_PER_W = BATCH // NW
BT = 128
K_BLOCKS = B_PER_W // BT
ROWS_PER_BLOCK = BT * N_FIELDS
STREAM = 128
S_PER_BLOCK = ROWS_PER_BLOCK // STREAM
FIRE = 13
IDX_ROWS = K_BLOCKS * S_PER_BLOCK

_mesh = plsc.VectorSubcoreMesh(
    core_axis_name="c", subcore_axis_name="s",
    num_cores=NUM_CORES, num_subcores=NUM_SUBCORES,
)


@functools.partial(
    pl.kernel,
    out_type=jax.ShapeDtypeStruct(
        (N_FIELDS, EMBED_DIM // 8, BATCH // BT, 8, BT), jnp.float32),
    mesh=_mesh,
    compiler_params=pltpu.CompilerParams(
        use_tc_tiling_on_sc=False, needs_layout_passes=False),
    scratch_types=[
        pltpu.VMEM((IDX_ROWS, STREAM), jnp.int32),
        pltpu.VMEM((2, ROWS_PER_BLOCK // 2, EMBED_DIM), jnp.float32),
        pltpu.VMEM((N_FIELDS, EMBED_DIM // 8, 8, BT), jnp.float32),
        pltpu.SemaphoreType.DMA,
        pltpu.SemaphoreType.DMA,
        pltpu.SemaphoreType.DMA,
    ],
)
def _gather_rows(idx_hbm, table_hbm, out_hbm, idx_v, rows_v, stage_v,
                 sem_a, sem_b, sem_out):
    wid = lax.axis_index("s") * NUM_CORES + lax.axis_index("c")
    pltpu.sync_copy(idx_hbm.at[pl.ds(wid * IDX_ROWS, IDX_ROWS)], idx_v)
    lane = jnp.arange(16, dtype=jnp.int32)
    lane26 = [lane * N_FIELDS + h * 16 * N_FIELDS for h in range(BT // 16)]
    sems = (sem_a, sem_b)
    HALF_S = FIRE
    N_HALF = K_BLOCKS * 2

    def fire(m):
        buf = rows_v.at[m % 2]
        descs = []
        for t in range(HALF_S):
            descs.append(
                pltpu.async_copy(
                    table_hbm.at[idx_v.at[m * HALF_S + t]],
                    buf.at[pl.ds(t * STREAM, STREAM)],
                    sems[m % 2],
                )
            )
        return descs

    def shuffle(m):
        q = m % 2
        buf = rows_v.at[m % 2]

        @plsc.parallel_loop(0, N_FIELDS, unroll=2)
        def _field(f):
            for e in range(EMBED_DIM):
                et, es = divmod(e, 8)
                col = lane * 0 + e
                for h in range(BT // 32):
                    v = plsc.load_gather(buf, [lane26[h] + f, col])
                    stage_v[f, et, es, pl.ds(q * 64 + h * 16, 16)] = v

    inflight = fire(0)
    out_desc = None
    for m in range(N_HALF):
        nxt = fire(m + 1) if m + 1 < N_HALF else []
        for c in inflight:
            c.wait()
        inflight = nxt
        if m % 2 == 0 and out_desc is not None:
            out_desc.wait()
            out_desc = None
        shuffle(m)
        if m % 2 == 1:
            k = m // 2
            out_desc = pltpu.async_copy(
                stage_v, out_hbm.at[:, :, wid * K_BLOCKS + k], sem_out)
    out_desc.wait()


def kernel(inputs, table):
    idx = inputs.reshape(BATCH * N_FIELDS // STREAM, STREAM)
    out5 = _gather_rows(idx, table)
    return out5.transpose(2, 4, 0, 1, 3).reshape(BATCH, N_FIELDS, EMBED_DIM)


# speedup vs baseline: 1.0710x; 1.0039x over previous
import functools

import jax
import jax.numpy as jnp
from jax import lax
from jax.experimental import pallas as pl
from jax.experimental.pallas import tpu as pltpu
from jax.experimental.pallas import tpu_sc as plsc

BATCH = 16384
N_FIELDS = 26
EMBED_DIM = 16

NUM_CORES = 2
NUM_SUBCORES = 16
NW = NUM_CORES * NUM_SUBCORES

B_PER_W = BATCH // NW
BT = 128
K_BLOCKS = B_PER_W // BT
ROWS_PER_BLOCK = BT * N_FIELDS
STREAM = 128
S_PER_BLOCK = ROWS_PER_BLOCK // STREAM
FIRE = 13
IDX_ROWS = K_BLOCKS * S_PER_BLOCK

_mesh = plsc.VectorSubcoreMesh(
    core_axis_name="c", subcore_axis_name="s",
    num_cores=NUM_CORES, num_subcores=NUM_SUBCORES,
)


@functools.partial(
    pl.kernel,
    out_type=jax.ShapeDtypeStruct(
        (N_FIELDS, EMBED_DIM // 8, BATCH // BT, 8, BT), jnp.float32),
    mesh=_mesh,
    compiler_params=pltpu.CompilerParams(
        use_tc_tiling_on_sc=False, needs_layout_passes=False),
    scratch_types=[
        pltpu.VMEM((IDX_ROWS, STREAM), jnp.int32),
        pltpu.VMEM((2, ROWS_PER_BLOCK // 2, EMBED_DIM), jnp.float32),
        pltpu.VMEM((N_FIELDS, EMBED_DIM // 8, 8, BT), jnp.float32),
        pltpu.SemaphoreType.DMA,
        pltpu.SemaphoreType.DMA,
    ],
)
def _gather_rows(idx_hbm, table_hbm, out_hbm, idx_v, rows_v, stage_v,
                 sem_a, sem_b):
    wid = lax.axis_index("s") * NUM_CORES + lax.axis_index("c")
    pltpu.sync_copy(idx_hbm.at[pl.ds(wid * IDX_ROWS, IDX_ROWS)], idx_v)
    lane = jnp.arange(16, dtype=jnp.int32)
    lane26 = [lane * N_FIELDS + h * 16 * N_FIELDS for h in range(BT // 16)]
    sems = (sem_a, sem_b)
    HALF_S = FIRE
    N_HALF = K_BLOCKS * 2

    def fire(m):
        buf = rows_v.at[m % 2]
        descs = []
        for t in range(HALF_S):
            descs.append(
                pltpu.async_copy(
                    table_hbm.at[idx_v.at[m * HALF_S + t]],
                    buf.at[pl.ds(t * STREAM, STREAM)],
                    sems[m % 2],
                )
            )
        return descs

    def shuffle(m):
        q = m % 2
        buf = rows_v.at[m % 2]

        @plsc.parallel_loop(0, N_FIELDS, unroll=2)
        def _field(f):
            for e in range(EMBED_DIM):
                et, es = divmod(e, 8)
                col = lane * 0 + e
                for h in range(BT // 32):
                    v = plsc.load_gather(buf, [lane26[h] + f, col])
                    stage_v[f, et, es, pl.ds(q * 64 + h * 16, 16)] = v

    inflight = fire(0)
    for m in range(N_HALF):
        nxt = fire(m + 1) if m + 1 < N_HALF else []
        for c in inflight:
            c.wait()
        inflight = nxt
        shuffle(m)
        if m % 2 == 1:
            k = m // 2
            pltpu.sync_copy(stage_v, out_hbm.at[:, :, wid * K_BLOCKS + k])


def kernel(inputs, table):
    idx = inputs.reshape(BATCH * N_FIELDS // STREAM, STREAM)
    out5 = _gather_rows(idx, table)
    return out5.transpose(2, 4, 0, 1, 3).reshape(BATCH, N_FIELDS, EMBED_DIM)
